# Initial kernel scaffold; baseline (speedup 1.0000x reference)
import jax
import jax.numpy as jnp
from jax.experimental import pallas as pl


def kernel(x, edge_index, W1, b1, W2, b2):
    raise NotImplementedError("write your pallas kernel here")


# speedup vs baseline: 38.0321x; 38.0321x over previous
import functools

import jax
import jax.numpy as jnp
from jax import lax
from jax.experimental import pallas as pl
from jax.experimental.pallas import tpu as pltpu
from jax.experimental.pallas import tpu_sc as plsc

N = 10000
NPAD = 10240
F = 16
F2 = 64
NC = 2
NS = 16
NW = NC * NS
L = 16
CHUNK = 128
KCH = 80
KF = 8
EP = NW * KCH * CHUNK
RPT = NPAD // NS
PADI = N + 16

_mesh = plsc.VectorSubcoreMesh(
    core_axis_name="c", subcore_axis_name="s", num_cores=NC, num_subcores=NS
)


def _fill1d(ref, n, val):
    def body(i, _):
        ref[pl.ds(i * L, L)] = jnp.full((L,), val, jnp.float32)
        return 0

    lax.fori_loop(0, n // L, body, 0)


@functools.partial(
    pl.kernel,
    out_type=jax.ShapeDtypeStruct((NC, NPAD), jnp.float32),
    mesh=_mesh,
    scratch_types=[
        pltpu.VMEM((KCH, CHUNK), jnp.int32),
        pltpu.VMEM((CHUNK,), jnp.float32),
        pltpu.VMEM((RPT,), jnp.float32),
        pltpu.VMEM_SHARED((NPAD,), jnp.float32),
        pltpu.SemaphoreType.DMA,
    ],
)
def _deg_kernel(col_hbm, out_hbm, colbuf, ones_v, zseg, acc_sh, sem):
    c = lax.axis_index("c")
    s = lax.axis_index("s")
    wid = s * NC + c
    _fill1d(ones_v, CHUNK, 1.0)
    _fill1d(zseg, RPT, 0.0)
    pltpu.sync_copy(zseg, acc_sh.at[pl.ds(s * RPT, RPT)])
    pltpu.sync_copy(col_hbm.at[pl.ds(wid * KCH, KCH), :], colbuf)
    plsc.subcore_barrier()

    def body(t, _):
        ds = []
        for i in range(KF):
            j = t * KF + i
            ds.append(pltpu.async_copy(ones_v, acc_sh.at[colbuf.at[j]], sem, add=True))
        for d in ds:
            d.wait()
        return 0

    lax.fori_loop(0, KCH // KF, body, 0)
    plsc.subcore_barrier()
    pltpu.sync_copy(acc_sh.at[pl.ds(s * RPT, RPT)], out_hbm.at[c, pl.ds(s * RPT, RPT)])


@functools.partial(
    pl.kernel,
    out_type=jax.ShapeDtypeStruct((NC, NPAD, F), jnp.float32),
    mesh=_mesh,
    scratch_types=[
        pltpu.VMEM((KCH, CHUNK), jnp.int32),
        pltpu.VMEM((KCH, CHUNK), jnp.int32),
        pltpu.VMEM((KF, CHUNK, F), jnp.float32),
        pltpu.VMEM((RPT, F), jnp.float32),
        pltpu.VMEM_SHARED((NPAD, F), jnp.float32),
        pltpu.SemaphoreType.DMA,
    ],
    compiler_params=pltpu.CompilerParams(use_tc_tiling_on_sc=False),
)
def _agg_kernel(y_hbm, row_hbm, col_hbm, out_hbm, rowbuf, colbuf, msgbuf, zseg,
                acc_sh, sem):
    c = lax.axis_index("c")
    s = lax.axis_index("s")
    wid = s * NC + c

    def zbody(i, _):
        zseg[i, :] = jnp.zeros((F,), jnp.float32)
        return 0

    lax.fori_loop(0, RPT, zbody, 0)
    pltpu.sync_copy(zseg, acc_sh.at[pl.ds(s * RPT, RPT), :])
    pltpu.sync_copy(row_hbm.at[pl.ds(wid * KCH, KCH), :], rowbuf)
    pltpu.sync_copy(col_hbm.at[pl.ds(wid * KCH, KCH), :], colbuf)
    plsc.subcore_barrier()

    def body(t, _):
        ds = []
        for i in range(KF):
            j = t * KF + i
            ds.append(pltpu.async_copy(y_hbm.at[rowbuf.at[j]], msgbuf.at[i], sem))
        for d in ds:
            d.wait()
        ds2 = []
        for i in range(KF):
            j = t * KF + i
            ds2.append(
                pltpu.async_copy(msgbuf.at[i], acc_sh.at[colbuf.at[j]], sem, add=True)
            )
        for d in ds2:
            d.wait()
        return 0

    lax.fori_loop(0, KCH // KF, body, 0)
    plsc.subcore_barrier()
    pltpu.sync_copy(
        acc_sh.at[pl.ds(s * RPT, RPT), :], out_hbm.at[c, pl.ds(s * RPT, RPT), :]
    )


def _tc1_body(xp_ref, degp_ref, w1_ref, y1_ref, dis_ref):
    deg = degp_ref[0] + degp_ref[1] + 1.0
    dis = lax.rsqrt(deg)
    dis2 = dis[:, None]
    xw = jnp.dot(xp_ref[...], w1_ref[...], preferred_element_type=jnp.float32)
    y1_ref[...] = xw * dis2
    dis_ref[...] = dis2


def _tc2_body(s1p_ref, y1_ref, dis_ref, b1_ref, g_ref):
    stot = s1p_ref[0] + s1p_ref[1] + y1_ref[...]
    dis2 = dis_ref[...]
    h = jnp.maximum(stot * dis2 + b1_ref[...], 0.0)
    g_ref[...] = h * dis2


def _tc3_body(s2p_ref, g_ref, dis_ref, w2_ref, b2_ref, out_ref):
    t = (s2p_ref[0] + s2p_ref[1] + g_ref[...]) * dis_ref[...]
    o = jnp.dot(t, w2_ref[...], preferred_element_type=jnp.float32) + b2_ref[...]
    m = jnp.max(o, axis=1, keepdims=True)
    e = o - m
    lse = jnp.log(jnp.sum(jnp.exp(e), axis=1, keepdims=True))
    out_ref[...] = e - lse


def kernel(x, edge_index, W1, b1, W2, b2):
    ei = edge_index.astype(jnp.int32)
    row, col = ei[0], ei[1]
    e = row.shape[0]
    pad = EP - e
    row2 = jnp.concatenate([row, jnp.full((pad,), PADI, jnp.int32)])
    col2 = jnp.concatenate([col, jnp.full((pad,), PADI, jnp.int32)])
    row2 = row2.reshape(EP // CHUNK, CHUNK)
    col2 = col2.reshape(EP // CHUNK, CHUNK)
    xp = jnp.pad(x, ((0, NPAD - N), (0, 0)))

    degp = _deg_kernel(col2)
    y1, dis = pl.pallas_call(
        _tc1_body,
        out_shape=(
            jax.ShapeDtypeStruct((NPAD, F), jnp.float32),
            jax.ShapeDtypeStruct((NPAD, 1), jnp.float32),
        ),
    )(xp, degp, W1)
    s1p = _agg_kernel(y1, row2, col2)
    g = pl.pallas_call(
        _tc2_body, out_shape=jax.ShapeDtypeStruct((NPAD, F), jnp.float32)
    )(s1p, y1, dis, b1.reshape(1, F))
    s2p = _agg_kernel(g, row2, col2)
    out = pl.pallas_call(
        _tc3_body, out_shape=jax.ShapeDtypeStruct((NPAD, F2), jnp.float32)
    )(s2p, g, dis, W2, b2.reshape(1, F2))
    return out[:N]


# speedup vs baseline: 40.1583x; 1.0559x over previous
import functools

import jax
import jax.numpy as jnp
from jax import lax
from jax.experimental import pallas as pl
from jax.experimental.pallas import tpu as pltpu
from jax.experimental.pallas import tpu_sc as plsc

N = 10000
NPAD = 10240
F = 16
F2 = 64
NC = 2
NS = 16
NW = NC * NS
L = 16
CHUNK = 128
KCH = 80
KF = 8
EP = NW * KCH * CHUNK
RPT = NPAD // NS
PADI = N + 16

_mesh = plsc.VectorSubcoreMesh(
    core_axis_name="c", subcore_axis_name="s", num_cores=NC, num_subcores=NS
)


def _fill1d(ref, n, val):
    def body(i, _):
        ref[pl.ds(i * L, L)] = jnp.full((L,), val, jnp.float32)
        return 0

    lax.fori_loop(0, n // L, body, 0)


@functools.partial(
    pl.kernel,
    out_type=jax.ShapeDtypeStruct((NC, NPAD), jnp.float32),
    mesh=_mesh,
    scratch_types=[
        pltpu.VMEM((KCH, CHUNK), jnp.int32),
        pltpu.VMEM((CHUNK,), jnp.float32),
        pltpu.VMEM((RPT,), jnp.float32),
        pltpu.VMEM_SHARED((NPAD,), jnp.float32),
        pltpu.SemaphoreType.DMA,
    ],
)
def _deg_kernel(col_hbm, out_hbm, colbuf, ones_v, zseg, acc_sh, sem):
    c = lax.axis_index("c")
    s = lax.axis_index("s")
    wid = s * NC + c
    _fill1d(ones_v, CHUNK, 1.0)
    _fill1d(zseg, RPT, 0.0)
    pltpu.sync_copy(zseg, acc_sh.at[pl.ds(s * RPT, RPT)])
    pltpu.sync_copy(col_hbm.at[pl.ds(wid * KCH, KCH), :], colbuf)
    plsc.subcore_barrier()

    def body(t, _):
        ds = []
        for i in range(KF):
            j = t * KF + i
            ds.append(pltpu.async_copy(ones_v, acc_sh.at[colbuf.at[j]], sem, add=True))
        for d in ds:
            d.wait()
        return 0

    lax.fori_loop(0, KCH // KF, body, 0)
    plsc.subcore_barrier()
    pltpu.sync_copy(acc_sh.at[pl.ds(s * RPT, RPT)], out_hbm.at[c, pl.ds(s * RPT, RPT)])


KFA = 16
NB = KCH // KFA


@functools.partial(
    pl.kernel,
    out_type=jax.ShapeDtypeStruct((NC, NPAD, F), jnp.float32),
    mesh=_mesh,
    scratch_types=[
        pltpu.VMEM((KCH, CHUNK), jnp.int32),
        pltpu.VMEM((KCH, CHUNK), jnp.int32),
        pltpu.VMEM((2, KFA, CHUNK, F), jnp.float32),
        pltpu.VMEM((RPT, F), jnp.float32),
        pltpu.VMEM_SHARED((NPAD, F), jnp.float32),
        pltpu.SemaphoreType.DMA,
        pltpu.SemaphoreType.DMA,
        pltpu.SemaphoreType.DMA,
    ],
    compiler_params=pltpu.CompilerParams(use_tc_tiling_on_sc=False),
)
def _agg_kernel(y_hbm, row_hbm, col_hbm, out_hbm, rowbuf, colbuf, msgbuf, zseg,
                acc_sh, gsem, ssem0, ssem1):
    c = lax.axis_index("c")
    s = lax.axis_index("s")
    wid = s * NC + c

    def zbody(i, _):
        zseg[i, :] = jnp.zeros((F,), jnp.float32)
        return 0

    lax.fori_loop(0, RPT, zbody, 0)
    pltpu.sync_copy(zseg, acc_sh.at[pl.ds(s * RPT, RPT), :])
    pltpu.sync_copy(row_hbm.at[pl.ds(wid * KCH, KCH), :], rowbuf)
    pltpu.sync_copy(col_hbm.at[pl.ds(wid * KCH, KCH), :], colbuf)
    plsc.subcore_barrier()

    ssems = (ssem0, ssem1)

    def gathers(t):
        p = t % 2
        return [
            pltpu.async_copy(
                y_hbm.at[rowbuf.at[t * KFA + i]], msgbuf.at[p, i], gsem
            )
            for i in range(KFA)
        ]

    def scatters(t):
        p = t % 2
        return [
            pltpu.async_copy(
                msgbuf.at[p, i], acc_sh.at[colbuf.at[t * KFA + i]], ssems[p],
                add=True,
            )
            for i in range(KFA)
        ]

    sd = {}
    gd = gathers(0)
    for t in range(NB):
        for d in gd:
            d.wait()
        sd[t] = scatters(t)
        if t + 1 < NB:
            if t - 1 >= 0:
                for d in sd[t - 1]:
                    d.wait()
            gd = gathers(t + 1)
    for d in sd[NB - 2]:
        d.wait()
    for d in sd[NB - 1]:
        d.wait()
    plsc.subcore_barrier()
    pltpu.sync_copy(
        acc_sh.at[pl.ds(s * RPT, RPT), :], out_hbm.at[c, pl.ds(s * RPT, RPT), :]
    )


def _tc1_body(xp_ref, degp_ref, w1_ref, y1_ref, dis_ref):
    deg = degp_ref[0] + degp_ref[1] + 1.0
    dis = lax.rsqrt(deg)
    dis2 = dis[:, None]
    xw = jnp.dot(xp_ref[...], w1_ref[...], preferred_element_type=jnp.float32)
    y1_ref[...] = xw * dis2
    dis_ref[...] = dis2


def _tc2_body(s1p_ref, y1_ref, dis_ref, b1_ref, g_ref):
    stot = s1p_ref[0] + s1p_ref[1] + y1_ref[...]
    dis2 = dis_ref[...]
    h = jnp.maximum(stot * dis2 + b1_ref[...], 0.0)
    g_ref[...] = h * dis2


def _tc3_body(s2p_ref, g_ref, dis_ref, w2_ref, b2_ref, out_ref):
    t = (s2p_ref[0] + s2p_ref[1] + g_ref[...]) * dis_ref[...]
    o = jnp.dot(t, w2_ref[...], preferred_element_type=jnp.float32) + b2_ref[...]
    m = jnp.max(o, axis=1, keepdims=True)
    e = o - m
    lse = jnp.log(jnp.sum(jnp.exp(e), axis=1, keepdims=True))
    out_ref[...] = e - lse


def kernel(x, edge_index, W1, b1, W2, b2):
    ei = edge_index.astype(jnp.int32)
    row, col = ei[0], ei[1]
    e = row.shape[0]
    pad = EP - e
    row2 = jnp.concatenate([row, jnp.full((pad,), PADI, jnp.int32)])
    col2 = jnp.concatenate([col, jnp.full((pad,), PADI, jnp.int32)])
    row2 = row2.reshape(EP // CHUNK, CHUNK)
    col2 = col2.reshape(EP // CHUNK, CHUNK)
    xp = jnp.pad(x, ((0, NPAD - N), (0, 0)))

    degp = _deg_kernel(col2)
    y1, dis = pl.pallas_call(
        _tc1_body,
        out_shape=(
            jax.ShapeDtypeStruct((NPAD, F), jnp.float32),
            jax.ShapeDtypeStruct((NPAD, 1), jnp.float32),
        ),
    )(xp, degp, W1)
    s1p = _agg_kernel(y1, row2, col2)
    g = pl.pallas_call(
        _tc2_body, out_shape=jax.ShapeDtypeStruct((NPAD, F), jnp.float32)
    )(s1p, y1, dis, b1.reshape(1, F))
    s2p = _agg_kernel(g, row2, col2)
    out = pl.pallas_call(
        _tc3_body, out_shape=jax.ShapeDtypeStruct((NPAD, F2), jnp.float32)
    )(s2p, g, dis, W2, b2.reshape(1, F2))
    return out[:N]


# speedup vs baseline: 57.2261x; 1.4250x over previous
import functools

import jax
import jax.numpy as jnp
from jax import lax
from jax.experimental import pallas as pl
from jax.experimental.pallas import tpu as pltpu
from jax.experimental.pallas import tpu_sc as plsc

N = 10000
NPAD = 10240
F = 16
F2 = 64
NC = 2
NS = 16
NW = NC * NS
L = 16
CHUNK = 128
KCH = 80
KF = 8
EP = NW * KCH * CHUNK
RPT = NPAD // NS
PADI = N + 16

_mesh = plsc.VectorSubcoreMesh(
    core_axis_name="c", subcore_axis_name="s", num_cores=NC, num_subcores=NS
)


def _fill1d(ref, n, val):
    def body(i, _):
        ref[pl.ds(i * L, L)] = jnp.full((L,), val, jnp.float32)
        return 0

    lax.fori_loop(0, n // L, body, 0)


@functools.partial(
    pl.kernel,
    out_type=jax.ShapeDtypeStruct((NC, NPAD), jnp.float32),
    mesh=_mesh,
    scratch_types=[
        pltpu.VMEM((KCH, CHUNK), jnp.int32),
        pltpu.VMEM((CHUNK,), jnp.float32),
        pltpu.VMEM((RPT,), jnp.float32),
        pltpu.VMEM_SHARED((NPAD,), jnp.float32),
        pltpu.SemaphoreType.DMA,
    ],
)
def _deg_kernel(col_hbm, out_hbm, colbuf, ones_v, zseg, acc_sh, sem):
    c = lax.axis_index("c")
    s = lax.axis_index("s")
    wid = s * NC + c
    _fill1d(ones_v, CHUNK, 1.0)
    _fill1d(zseg, RPT, 0.0)
    pltpu.sync_copy(zseg, acc_sh.at[pl.ds(s * RPT, RPT)])
    pltpu.sync_copy(col_hbm.at[pl.ds(wid * KCH, KCH), :], colbuf)
    plsc.subcore_barrier()

    def body(t, _):
        ds = []
        for i in range(KF):
            j = t * KF + i
            ds.append(pltpu.async_copy(ones_v, acc_sh.at[colbuf.at[j]], sem, add=True))
        for d in ds:
            d.wait()
        return 0

    lax.fori_loop(0, KCH // KF, body, 0)
    plsc.subcore_barrier()
    pltpu.sync_copy(acc_sh.at[pl.ds(s * RPT, RPT)], out_hbm.at[c, pl.ds(s * RPT, RPT)])


KFA = 16
NB = KCH // KFA


@functools.partial(
    pl.kernel,
    out_type=jax.ShapeDtypeStruct((NC, NPAD, F), jnp.float32),
    mesh=_mesh,
    scratch_types=[
        pltpu.VMEM((KCH, CHUNK), jnp.int32),
        pltpu.VMEM((KCH, CHUNK), jnp.int32),
        pltpu.VMEM((2, KFA, CHUNK, F), jnp.float32),
        pltpu.VMEM((RPT, F), jnp.float32),
        pltpu.VMEM_SHARED((NPAD, F), jnp.float32),
        pltpu.VMEM_SHARED((NPAD, F), jnp.float32),
        pltpu.SemaphoreType.DMA,
        pltpu.SemaphoreType.DMA,
        pltpu.SemaphoreType.DMA,
    ],
    compiler_params=pltpu.CompilerParams(use_tc_tiling_on_sc=False),
)
def _agg_kernel(y_hbm, row_hbm, col_hbm, out_hbm, rowbuf, colbuf, msgbuf, zseg,
                acc_sh, y_sh, gsem, ssem0, ssem1):
    c = lax.axis_index("c")
    s = lax.axis_index("s")
    wid = s * NC + c

    def zbody(i, _):
        zseg[i, :] = jnp.zeros((F,), jnp.float32)
        return 0

    lax.fori_loop(0, RPT, zbody, 0)
    pltpu.sync_copy(zseg, acc_sh.at[pl.ds(s * RPT, RPT), :])
    pltpu.sync_copy(
        y_hbm.at[pl.ds(s * RPT, RPT), :], y_sh.at[pl.ds(s * RPT, RPT), :]
    )
    pltpu.sync_copy(row_hbm.at[pl.ds(wid * KCH, KCH), :], rowbuf)
    pltpu.sync_copy(col_hbm.at[pl.ds(wid * KCH, KCH), :], colbuf)
    plsc.subcore_barrier()

    ssems = (ssem0, ssem1)

    def gathers(t):
        p = t % 2
        return [
            pltpu.async_copy(
                y_sh.at[rowbuf.at[t * KFA + i]], msgbuf.at[p, i], gsem
            )
            for i in range(KFA)
        ]

    def scatters(t):
        p = t % 2
        return [
            pltpu.async_copy(
                msgbuf.at[p, i], acc_sh.at[colbuf.at[t * KFA + i]], ssems[p],
                add=True,
            )
            for i in range(KFA)
        ]

    sd = {}
    gd = gathers(0)
    for t in range(NB):
        for d in gd:
            d.wait()
        sd[t] = scatters(t)
        if t + 1 < NB:
            if t - 1 >= 0:
                for d in sd[t - 1]:
                    d.wait()
            gd = gathers(t + 1)
    for d in sd[NB - 2]:
        d.wait()
    for d in sd[NB - 1]:
        d.wait()
    plsc.subcore_barrier()
    pltpu.sync_copy(
        acc_sh.at[pl.ds(s * RPT, RPT), :], out_hbm.at[c, pl.ds(s * RPT, RPT), :]
    )


def _tc1_body(xp_ref, degp_ref, w1_ref, y1_ref, dis_ref):
    deg = degp_ref[0] + degp_ref[1] + 1.0
    dis = lax.rsqrt(deg)
    dis2 = dis[:, None]
    xw = jnp.dot(xp_ref[...], w1_ref[...], preferred_element_type=jnp.float32)
    y1_ref[...] = xw * dis2
    dis_ref[...] = dis2


def _tc2_body(s1p_ref, y1_ref, dis_ref, b1_ref, g_ref):
    stot = s1p_ref[0] + s1p_ref[1] + y1_ref[...]
    dis2 = dis_ref[...]
    h = jnp.maximum(stot * dis2 + b1_ref[...], 0.0)
    g_ref[...] = h * dis2


def _tc3_body(s2p_ref, g_ref, dis_ref, w2_ref, b2_ref, out_ref):
    t = (s2p_ref[0] + s2p_ref[1] + g_ref[...]) * dis_ref[...]
    o = jnp.dot(t, w2_ref[...], preferred_element_type=jnp.float32) + b2_ref[...]
    m = jnp.max(o, axis=1, keepdims=True)
    e = o - m
    lse = jnp.log(jnp.sum(jnp.exp(e), axis=1, keepdims=True))
    out_ref[...] = e - lse


def kernel(x, edge_index, W1, b1, W2, b2):
    ei = edge_index.astype(jnp.int32)
    row, col = ei[0], ei[1]
    e = row.shape[0]
    pad = EP - e
    row2 = jnp.concatenate([row, jnp.full((pad,), PADI, jnp.int32)])
    col2 = jnp.concatenate([col, jnp.full((pad,), PADI, jnp.int32)])
    row2 = row2.reshape(EP // CHUNK, CHUNK)
    col2 = col2.reshape(EP // CHUNK, CHUNK)
    xp = jnp.pad(x, ((0, NPAD - N), (0, 0)))

    degp = _deg_kernel(col2)
    y1, dis = pl.pallas_call(
        _tc1_body,
        out_shape=(
            jax.ShapeDtypeStruct((NPAD, F), jnp.float32),
            jax.ShapeDtypeStruct((NPAD, 1), jnp.float32),
        ),
    )(xp, degp, W1)
    s1p = _agg_kernel(y1, row2, col2)
    g = pl.pallas_call(
        _tc2_body, out_shape=jax.ShapeDtypeStruct((NPAD, F), jnp.float32)
    )(s1p, y1, dis, b1.reshape(1, F))
    s2p = _agg_kernel(g, row2, col2)
    out = pl.pallas_call(
        _tc3_body, out_shape=jax.ShapeDtypeStruct((NPAD, F2), jnp.float32)
    )(s2p, g, dis, W2, b2.reshape(1, F2))
    return out[:N]


# speedup vs baseline: 67.7026x; 1.1831x over previous
import functools

import jax
import jax.numpy as jnp
from jax import lax
from jax.experimental import pallas as pl
from jax.experimental.pallas import tpu as pltpu
from jax.experimental.pallas import tpu_sc as plsc

N = 10000
NPAD = 10240
F = 16
F2 = 64
NC = 2
NS = 16
NW = NC * NS
L = 16
CHUNK = 128
KCH = 80
KF = 8
KFA = 16
NB = KCH // KFA
EP = NW * KCH * CHUNK
RPT = NPAD // NS
PADI = N + 16

_mesh = plsc.VectorSubcoreMesh(
    core_axis_name="c", subcore_axis_name="s", num_cores=NC, num_subcores=NS
)
_sc_params = pltpu.CompilerParams(use_tc_tiling_on_sc=False)


def _fill1d(ref, n, val):
    def body(i, _):
        ref[pl.ds(i * L, L)] = jnp.full((L,), val, jnp.float32)
        return 0

    lax.fori_loop(0, n // L, body, 0)


def _vrsqrt(v):
    i = jax.lax.bitcast_convert_type(v, jnp.int32)
    i = jnp.int32(0x5F3759DF) - jax.lax.shift_right_logical(i, 1)
    y = jax.lax.bitcast_convert_type(i, jnp.float32)
    for _ in range(3):
        y = y * (1.5 - 0.5 * v * y * y)
    return y


@functools.partial(
    pl.kernel,
    out_type=jax.ShapeDtypeStruct((NC, NPAD), jnp.float32),
    mesh=_mesh,
    scratch_types=[
        pltpu.VMEM((KCH, CHUNK), jnp.int32),
        pltpu.VMEM((CHUNK,), jnp.float32),
        pltpu.VMEM((RPT,), jnp.float32),
        pltpu.VMEM_SHARED((NPAD,), jnp.float32),
        pltpu.SemaphoreType.DMA,
    ],
    compiler_params=_sc_params,
)
def _deg_kernel(col_hbm, out_hbm, colbuf, ones_v, zseg, acc_sh, sem):
    c = lax.axis_index("c")
    s = lax.axis_index("s")
    wid = s * NC + c
    _fill1d(ones_v, CHUNK, 1.0)
    _fill1d(zseg, RPT, 0.0)
    pltpu.sync_copy(zseg, acc_sh.at[pl.ds(s * RPT, RPT)])
    pltpu.sync_copy(col_hbm.at[pl.ds(wid * KCH, KCH), :], colbuf)
    plsc.subcore_barrier()

    def body(t, _):
        ds = []
        for i in range(KF):
            j = t * KF + i
            ds.append(pltpu.async_copy(ones_v, acc_sh.at[colbuf.at[j]], sem, add=True))
        for d in ds:
            d.wait()
        return 0

    lax.fori_loop(0, KCH // KF, body, 0)
    plsc.subcore_barrier()
    pltpu.sync_copy(acc_sh.at[pl.ds(s * RPT, RPT)], out_hbm.at[c, pl.ds(s * RPT, RPT)])


def _edge_pipeline(y_sh, acc_sh, rowbuf, colbuf, msgbuf, gsem, ssem0, ssem1):
    ssems = (ssem0, ssem1)

    def gathers(t):
        p = t % 2
        return [
            pltpu.async_copy(y_sh.at[rowbuf.at[t * KFA + i]], msgbuf.at[p, i], gsem)
            for i in range(KFA)
        ]

    def scatters(t):
        p = t % 2
        return [
            pltpu.async_copy(
                msgbuf.at[p, i], acc_sh.at[colbuf.at[t * KFA + i]], ssems[p], add=True
            )
            for i in range(KFA)
        ]

    sd = {}
    gd = gathers(0)
    for t in range(NB):
        for d in gd:
            d.wait()
        sd[t] = scatters(t)
        if t + 1 < NB:
            if t - 1 >= 0:
                for d in sd[t - 1]:
                    d.wait()
            gd = gathers(t + 1)
    for d in sd[NB - 2]:
        d.wait()
    for d in sd[NB - 1]:
        d.wait()


def _stage_and_init(c, s, seg, y_sh, acc_sh):
    sl = pl.ds(s * RPT, RPT)
    pltpu.sync_copy(seg, y_sh.at[sl, :])

    @pl.when(c == 0)
    def _():
        pltpu.sync_copy(seg, acc_sh.at[sl, :])

    @pl.when(c != 0)
    def _():
        def zb(i, _):
            seg[i, :] = jnp.zeros((F,), jnp.float32)
            return 0

        lax.fori_loop(0, RPT, zb, 0)
        pltpu.sync_copy(seg, acc_sh.at[sl, :])


@functools.partial(
    pl.kernel,
    out_type=(
        jax.ShapeDtypeStruct((NC, NPAD, F), jnp.float32),
        jax.ShapeDtypeStruct((NPAD,), jnp.float32),
    ),
    mesh=_mesh,
    scratch_types=[
        pltpu.VMEM((KCH, CHUNK), jnp.int32),
        pltpu.VMEM((KCH, CHUNK), jnp.int32),
        pltpu.VMEM((2, KFA, CHUNK, F), jnp.float32),
        pltpu.VMEM((RPT, F), jnp.float32),
        pltpu.VMEM((RPT,), jnp.float32),
        pltpu.VMEM((RPT,), jnp.float32),
        pltpu.VMEM((RPT,), jnp.float32),
        pltpu.VMEM_SHARED((NPAD, F), jnp.float32),
        pltpu.VMEM_SHARED((NPAD, F), jnp.float32),
        pltpu.SemaphoreType.DMA,
        pltpu.SemaphoreType.DMA,
        pltpu.SemaphoreType.DMA,
    ],
    compiler_params=_sc_params,
)
def _agg1_kernel(xw_hbm, degp_hbm, row_hbm, col_hbm, s1p_hbm, dis_hbm,
                 rowbuf, colbuf, msgbuf, seg, d0seg, d1seg, disseg,
                 acc_sh, y_sh, gsem, ssem0, ssem1):
    c = lax.axis_index("c")
    s = lax.axis_index("s")
    wid = s * NC + c
    sl = pl.ds(s * RPT, RPT)
    pltpu.sync_copy(xw_hbm.at[sl, :], seg)
    pltpu.sync_copy(degp_hbm.at[0, sl], d0seg)
    pltpu.sync_copy(degp_hbm.at[1, sl], d1seg)
    pltpu.sync_copy(row_hbm.at[pl.ds(wid * KCH, KCH), :], rowbuf)
    pltpu.sync_copy(col_hbm.at[pl.ds(wid * KCH, KCH), :], colbuf)

    def dbody(i, _):
        v = d0seg[pl.ds(i * L, L)] + d1seg[pl.ds(i * L, L)] + 1.0
        disseg[pl.ds(i * L, L)] = _vrsqrt(v)
        return 0

    lax.fori_loop(0, RPT // L, dbody, 0)

    def ybody(i, _):
        dv = disseg[pl.ds(i * L, L)]
        for k in range(L):
            r = i * L + k
            seg[r, :] = seg[r, :] * dv[k]
        return 0

    lax.fori_loop(0, RPT // L, ybody, 0)

    @pl.when(c == 0)
    def _():
        pltpu.sync_copy(disseg, dis_hbm.at[sl])

    _stage_and_init(c, s, seg, y_sh, acc_sh)
    plsc.subcore_barrier()
    _edge_pipeline(y_sh, acc_sh, rowbuf, colbuf, msgbuf, gsem, ssem0, ssem1)
    plsc.subcore_barrier()
    pltpu.sync_copy(acc_sh.at[sl, :], s1p_hbm.at[c, sl, :])


@functools.partial(
    pl.kernel,
    out_type=jax.ShapeDtypeStruct((NC, NPAD, F), jnp.float32),
    mesh=_mesh,
    scratch_types=[
        pltpu.VMEM((KCH, CHUNK), jnp.int32),
        pltpu.VMEM((KCH, CHUNK), jnp.int32),
        pltpu.VMEM((2, KFA, CHUNK, F), jnp.float32),
        pltpu.VMEM((RPT, F), jnp.float32),
        pltpu.VMEM((RPT, F), jnp.float32),
        pltpu.VMEM((RPT,), jnp.float32),
        pltpu.VMEM((F,), jnp.float32),
        pltpu.VMEM_SHARED((NPAD, F), jnp.float32),
        pltpu.VMEM_SHARED((NPAD, F), jnp.float32),
        pltpu.SemaphoreType.DMA,
        pltpu.SemaphoreType.DMA,
        pltpu.SemaphoreType.DMA,
    ],
    compiler_params=_sc_params,
)
def _agg2_kernel(s1p_hbm, dis_hbm, b1_hbm, row_hbm, col_hbm, s2p_hbm,
                 rowbuf, colbuf, msgbuf, seg, p1seg, disseg, b1v,
                 acc_sh, y_sh, gsem, ssem0, ssem1):
    c = lax.axis_index("c")
    s = lax.axis_index("s")
    wid = s * NC + c
    sl = pl.ds(s * RPT, RPT)
    pltpu.sync_copy(s1p_hbm.at[0, sl, :], seg)
    pltpu.sync_copy(s1p_hbm.at[1, sl, :], p1seg)
    pltpu.sync_copy(dis_hbm.at[sl], disseg)
    pltpu.sync_copy(b1_hbm, b1v)
    pltpu.sync_copy(row_hbm.at[pl.ds(wid * KCH, KCH), :], rowbuf)
    pltpu.sync_copy(col_hbm.at[pl.ds(wid * KCH, KCH), :], colbuf)
    b1r = b1v[...]

    def gbody(i, _):
        dv = disseg[pl.ds(i * L, L)]
        for k in range(L):
            r = i * L + k
            d = dv[k]
            h = jnp.maximum((seg[r, :] + p1seg[r, :]) * d + b1r, 0.0)
            seg[r, :] = h * d
        return 0

    lax.fori_loop(0, RPT // L, gbody, 0)
    _stage_and_init(c, s, seg, y_sh, acc_sh)
    plsc.subcore_barrier()
    _edge_pipeline(y_sh, acc_sh, rowbuf, colbuf, msgbuf, gsem, ssem0, ssem1)
    plsc.subcore_barrier()
    pltpu.sync_copy(acc_sh.at[sl, :], s2p_hbm.at[c, sl, :])


def _tcmm_body(xp_ref, w1_ref, xw_ref):
    xw_ref[...] = jnp.dot(
        xp_ref[...], w1_ref[...], preferred_element_type=jnp.float32
    )


def _tc3_body(s2p_ref, dis_ref, w2_ref, b2_ref, out_ref):
    t = (s2p_ref[0] + s2p_ref[1]) * dis_ref[...][:, None]
    o = jnp.dot(t, w2_ref[...], preferred_element_type=jnp.float32) + b2_ref[...]
    m = jnp.max(o, axis=1, keepdims=True)
    e = o - m
    lse = jnp.log(jnp.sum(jnp.exp(e), axis=1, keepdims=True))
    out_ref[...] = e - lse


def kernel(x, edge_index, W1, b1, W2, b2):
    ei = edge_index.astype(jnp.int32)
    row, col = ei[0], ei[1]
    e = row.shape[0]
    pad = EP - e
    row2 = jnp.concatenate([row, jnp.full((pad,), PADI, jnp.int32)])
    col2 = jnp.concatenate([col, jnp.full((pad,), PADI, jnp.int32)])
    row2 = row2.reshape(EP // CHUNK, CHUNK)
    col2 = col2.reshape(EP // CHUNK, CHUNK)
    xp = jnp.pad(x, ((0, NPAD - N), (0, 0)))

    degp = _deg_kernel(col2)
    xw = pl.pallas_call(
        _tcmm_body, out_shape=jax.ShapeDtypeStruct((NPAD, F), jnp.float32)
    )(xp, W1)
    s1p, dis = _agg1_kernel(xw, degp, row2, col2)
    s2p = _agg2_kernel(s1p, dis, b1, row2, col2)
    out = pl.pallas_call(
        _tc3_body, out_shape=jax.ShapeDtypeStruct((NPAD, F2), jnp.float32)
    )(s2p, dis, W2, b2.reshape(1, F2))
    return out[:N]


# speedup vs baseline: 72.9555x; 1.0776x over previous
import functools

import jax
import jax.numpy as jnp
from jax import lax
from jax.experimental import pallas as pl
from jax.experimental.pallas import tpu as pltpu
from jax.experimental.pallas import tpu_sc as plsc

N = 10000
NPAD = 10240
F = 16
F2 = 64
NC = 2
NS = 16
NW = NC * NS
L = 16
CHUNK = 128
KCH = 80
KF = 8
KFA = 8
KC0 = 96
KC1 = 64
KD0 = 112
KD1 = 48
EP = NW * KCH * CHUNK
RPT = NPAD // NS
PADI = N + 16

_mesh = plsc.VectorSubcoreMesh(
    core_axis_name="c", subcore_axis_name="s", num_cores=NC, num_subcores=NS
)
_sc_params = pltpu.CompilerParams(use_tc_tiling_on_sc=False)


def _fill1d(ref, n, val):
    def body(i, _):
        ref[pl.ds(i * L, L)] = jnp.full((L,), val, jnp.float32)
        return 0

    lax.fori_loop(0, n // L, body, 0)


def _vrsqrt(v):
    i = jax.lax.bitcast_convert_type(v, jnp.int32)
    i = jnp.int32(0x5F3759DF) - jax.lax.shift_right_logical(i, 1)
    y = jax.lax.bitcast_convert_type(i, jnp.float32)
    for _ in range(3):
        y = y * (1.5 - 0.5 * v * y * y)
    return y


@functools.partial(
    pl.kernel,
    out_type=jax.ShapeDtypeStruct((NC, NPAD), jnp.float32),
    mesh=_mesh,
    scratch_types=[
        pltpu.VMEM((KD0, CHUNK), jnp.int32),
        pltpu.VMEM((CHUNK,), jnp.float32),
        pltpu.VMEM((RPT,), jnp.float32),
        pltpu.VMEM_SHARED((NPAD,), jnp.float32),
        pltpu.SemaphoreType.DMA,
    ],
    compiler_params=_sc_params,
)
def _deg_kernel(col_hbm, out_hbm, colbuf, ones_v, zseg, acc_sh, sem):
    c = lax.axis_index("c")
    s = lax.axis_index("s")
    _fill1d(ones_v, CHUNK, 1.0)
    _fill1d(zseg, RPT, 0.0)
    pltpu.sync_copy(zseg, acc_sh.at[pl.ds(s * RPT, RPT)])

    @pl.when(c == 0)
    def _():
        pltpu.sync_copy(col_hbm.at[pl.ds(s * KD0, KD0), :], colbuf)

    @pl.when(c != 0)
    def _():
        pltpu.sync_copy(
            col_hbm.at[pl.ds(NS * KD0 + s * KD1, KD1), :],
            colbuf.at[pl.ds(0, KD1), :],
        )

    plsc.subcore_barrier()
    nt = jnp.where(c == 0, KD0 // KF, KD1 // KF)

    def body(t, _):
        ds = []
        for i in range(KF):
            j = t * KF + i
            ds.append(pltpu.async_copy(ones_v, acc_sh.at[colbuf.at[j]], sem, add=True))
        for d in ds:
            d.wait()
        return 0

    lax.fori_loop(0, nt, body, 0)
    plsc.subcore_barrier()
    pltpu.sync_copy(acc_sh.at[pl.ds(s * RPT, RPT)], out_hbm.at[c, pl.ds(s * RPT, RPT)])


def _edge_pipeline(nb, y_sh, acc_sh, rowbuf, colbuf, msgbuf, gsem, ssem0, ssem1):
    ssems = (ssem0, ssem1)

    def issue_g(t, p):
        for i in range(KFA):
            pltpu.async_copy(y_sh.at[rowbuf.at[t * KFA + i]], msgbuf.at[p, i], gsem)

    def wait_g(p):
        for i in range(KFA):
            pltpu.make_async_copy(
                y_sh.at[rowbuf.at[i]], msgbuf.at[p, i], gsem
            ).wait()

    def issue_s(t, p):
        for i in range(KFA):
            pltpu.async_copy(
                msgbuf.at[p, i], acc_sh.at[colbuf.at[t * KFA + i]], ssems[p],
                add=True,
            )

    def wait_s(p):
        for i in range(KFA):
            pltpu.make_async_copy(
                msgbuf.at[p, i], acc_sh.at[colbuf.at[i]], ssems[p]
            ).wait()

    issue_g(0, 0)

    def pair(u, _):
        t = 2 * u
        wait_g(0)
        issue_s(t, 0)

        @pl.when(u >= 1)
        def _():
            wait_s(1)
        issue_g(t + 1, 1)
        wait_g(1)
        issue_s(t + 1, 1)

        @pl.when(t + 2 < nb)
        def _():
            wait_s(0)
            issue_g(t + 2, 0)

        return 0

    lax.fori_loop(0, nb // 2, pair, 0)
    wait_s(0)
    wait_s(1)


def _stage_and_init(c, s, seg, y_sh, acc_sh):
    sl = pl.ds(s * RPT, RPT)
    pltpu.sync_copy(seg, y_sh.at[sl, :])

    @pl.when(c == 0)
    def _():
        pltpu.sync_copy(seg, acc_sh.at[sl, :])

    @pl.when(c != 0)
    def _():
        def zb(i, _):
            seg[i, :] = jnp.zeros((F,), jnp.float32)
            return 0

        lax.fori_loop(0, RPT, zb, 0)
        pltpu.sync_copy(seg, acc_sh.at[sl, :])


@functools.partial(
    pl.kernel,
    out_type=(
        jax.ShapeDtypeStruct((NC, NPAD, F), jnp.float32),
        jax.ShapeDtypeStruct((NPAD,), jnp.float32),
    ),
    mesh=_mesh,
    scratch_types=[
        pltpu.VMEM((KC0, CHUNK), jnp.int32),
        pltpu.VMEM((KC0, CHUNK), jnp.int32),
        pltpu.VMEM((2, KFA, CHUNK, F), jnp.float32),
        pltpu.VMEM((RPT, F), jnp.float32),
        pltpu.VMEM((RPT,), jnp.float32),
        pltpu.VMEM((RPT,), jnp.float32),
        pltpu.VMEM((RPT,), jnp.float32),
        pltpu.VMEM_SHARED((NPAD, F), jnp.float32),
        pltpu.VMEM_SHARED((NPAD, F), jnp.float32),
        pltpu.SemaphoreType.DMA,
        pltpu.SemaphoreType.DMA,
        pltpu.SemaphoreType.DMA,
    ],
    compiler_params=_sc_params,
)
def _agg1_kernel(xw_hbm, degp_hbm, row_hbm, col_hbm, s1p_hbm, dis_hbm,
                 rowbuf, colbuf, msgbuf, seg, d0seg, d1seg, disseg,
                 acc_sh, y_sh, gsem, ssem0, ssem1):
    c = lax.axis_index("c")
    s = lax.axis_index("s")
    wid = s * NC + c
    sl = pl.ds(s * RPT, RPT)
    pltpu.sync_copy(xw_hbm.at[sl, :], seg)
    pltpu.sync_copy(degp_hbm.at[0, sl], d0seg)
    pltpu.sync_copy(degp_hbm.at[1, sl], d1seg)
    @pl.when(c == 0)
    def _():
        pltpu.sync_copy(row_hbm.at[pl.ds(s * KC0, KC0), :], rowbuf)
        pltpu.sync_copy(col_hbm.at[pl.ds(s * KC0, KC0), :], colbuf)

    @pl.when(c != 0)
    def _():
        base = NS * KC0 + s * KC1
        pltpu.sync_copy(row_hbm.at[pl.ds(base, KC1), :], rowbuf.at[pl.ds(0, KC1), :])
        pltpu.sync_copy(col_hbm.at[pl.ds(base, KC1), :], colbuf.at[pl.ds(0, KC1), :])

    def dbody(i, _):
        v = d0seg[pl.ds(i * L, L)] + d1seg[pl.ds(i * L, L)] + 1.0
        disseg[pl.ds(i * L, L)] = _vrsqrt(v)
        return 0

    lax.fori_loop(0, RPT // L, dbody, 0)

    def ybody(i, _):
        dv = disseg[pl.ds(i * L, L)]
        for k in range(L):
            r = i * L + k
            seg[r, :] = seg[r, :] * dv[k]
        return 0

    lax.fori_loop(0, RPT // L, ybody, 0)

    @pl.when(c == 0)
    def _():
        pltpu.sync_copy(disseg, dis_hbm.at[sl])

    _stage_and_init(c, s, seg, y_sh, acc_sh)
    plsc.subcore_barrier()
    nb = jnp.where(c == 0, KC0 // KFA, KC1 // KFA)
    _edge_pipeline(nb, y_sh, acc_sh, rowbuf, colbuf, msgbuf, gsem, ssem0, ssem1)
    plsc.subcore_barrier()
    pltpu.sync_copy(acc_sh.at[sl, :], s1p_hbm.at[c, sl, :])


@functools.partial(
    pl.kernel,
    out_type=jax.ShapeDtypeStruct((NC, NPAD, F), jnp.float32),
    mesh=_mesh,
    scratch_types=[
        pltpu.VMEM((KC0, CHUNK), jnp.int32),
        pltpu.VMEM((KC0, CHUNK), jnp.int32),
        pltpu.VMEM((2, KFA, CHUNK, F), jnp.float32),
        pltpu.VMEM((RPT, F), jnp.float32),
        pltpu.VMEM((RPT, F), jnp.float32),
        pltpu.VMEM((RPT,), jnp.float32),
        pltpu.VMEM((F,), jnp.float32),
        pltpu.VMEM_SHARED((NPAD, F), jnp.float32),
        pltpu.VMEM_SHARED((NPAD, F), jnp.float32),
        pltpu.SemaphoreType.DMA,
        pltpu.SemaphoreType.DMA,
        pltpu.SemaphoreType.DMA,
    ],
    compiler_params=_sc_params,
)
def _agg2_kernel(s1p_hbm, dis_hbm, b1_hbm, row_hbm, col_hbm, s2p_hbm,
                 rowbuf, colbuf, msgbuf, seg, p1seg, disseg, b1v,
                 acc_sh, y_sh, gsem, ssem0, ssem1):
    c = lax.axis_index("c")
    s = lax.axis_index("s")
    wid = s * NC + c
    sl = pl.ds(s * RPT, RPT)
    pltpu.sync_copy(s1p_hbm.at[0, sl, :], seg)
    pltpu.sync_copy(s1p_hbm.at[1, sl, :], p1seg)
    pltpu.sync_copy(dis_hbm.at[sl], disseg)
    pltpu.sync_copy(b1_hbm, b1v)
    @pl.when(c == 0)
    def _():
        pltpu.sync_copy(row_hbm.at[pl.ds(s * KC0, KC0), :], rowbuf)
        pltpu.sync_copy(col_hbm.at[pl.ds(s * KC0, KC0), :], colbuf)

    @pl.when(c != 0)
    def _():
        base = NS * KC0 + s * KC1
        pltpu.sync_copy(row_hbm.at[pl.ds(base, KC1), :], rowbuf.at[pl.ds(0, KC1), :])
        pltpu.sync_copy(col_hbm.at[pl.ds(base, KC1), :], colbuf.at[pl.ds(0, KC1), :])
    b1r = b1v[...]

    def gbody(i, _):
        dv = disseg[pl.ds(i * L, L)]
        for k in range(L):
            r = i * L + k
            d = dv[k]
            h = jnp.maximum((seg[r, :] + p1seg[r, :]) * d + b1r, 0.0)
            seg[r, :] = h * d
        return 0

    lax.fori_loop(0, RPT // L, gbody, 0)
    _stage_and_init(c, s, seg, y_sh, acc_sh)
    plsc.subcore_barrier()
    nb = jnp.where(c == 0, KC0 // KFA, KC1 // KFA)
    _edge_pipeline(nb, y_sh, acc_sh, rowbuf, colbuf, msgbuf, gsem, ssem0, ssem1)
    plsc.subcore_barrier()
    pltpu.sync_copy(acc_sh.at[sl, :], s2p_hbm.at[c, sl, :])


def _tcmm_body(xp_ref, w1_ref, xw_ref):
    xw_ref[...] = jnp.dot(
        xp_ref[...], w1_ref[...], preferred_element_type=jnp.float32
    )


def _tc3_body(s2p_ref, dis_ref, w2_ref, b2_ref, out_ref):
    t = (s2p_ref[0] + s2p_ref[1]) * dis_ref[...][:, None]
    o = jnp.dot(t, w2_ref[...], preferred_element_type=jnp.float32) + b2_ref[...]
    m = jnp.max(o, axis=1, keepdims=True)
    e = o - m
    lse = jnp.log(jnp.sum(jnp.exp(e), axis=1, keepdims=True))
    out_ref[...] = e - lse


def kernel(x, edge_index, W1, b1, W2, b2):
    ei = edge_index.astype(jnp.int32)
    row, col = ei[0], ei[1]
    e = row.shape[0]
    pad = EP - e
    row2 = jnp.concatenate([row, jnp.full((pad,), PADI, jnp.int32)])
    col2 = jnp.concatenate([col, jnp.full((pad,), PADI, jnp.int32)])
    row2 = row2.reshape(EP // CHUNK, CHUNK)
    col2 = col2.reshape(EP // CHUNK, CHUNK)
    xp = jnp.pad(x, ((0, NPAD - N), (0, 0)))

    degp = _deg_kernel(col2)
    xw = pl.pallas_call(
        _tcmm_body, out_shape=jax.ShapeDtypeStruct((NPAD, F), jnp.float32)
    )(xp, W1)
    s1p, dis = _agg1_kernel(xw, degp, row2, col2)
    s2p = _agg2_kernel(s1p, dis, b1, row2, col2)
    out = pl.pallas_call(
        _tc3_body, out_shape=jax.ShapeDtypeStruct((NPAD, F2), jnp.float32)
    )(s2p, dis, W2, b2.reshape(1, F2))
    return out[:N]
